# baseline (speedup 1.0000x reference)
import functools

import jax
import jax.numpy as jnp
from jax import lax
from jax.experimental import pallas as pl
from jax.experimental.pallas import tpu as pltpu
from jax.experimental.pallas import tpu_sc as plsc

N = 10000
E = 320000
D = 128
NP = 10240
NC = 2
NS = 16
NW = NC * NS
EPP = E // NW
HALF = NP // NC
NPC = 5376
TRASH = HALF
SEG = 256
SEGS = 40
KC = 64
FLUSH = 2048
RING = FLUSH + 16


def _bin_edges(srcf, dstf):
    mesh = plsc.VectorSubcoreMesh(core_axis_name="c", subcore_axis_name="s")

    @functools.partial(
        pl.kernel,
        out_type=[
            jax.ShapeDtypeStruct((NC, NW, SEGS * SEG), jnp.int32),
            jax.ShapeDtypeStruct((NC, NW, SEGS * SEG), jnp.int32),
            jax.ShapeDtypeStruct((NW, 16), jnp.int32),
            jax.ShapeDtypeStruct((NW, NP), jnp.float32),
        ],
        mesh=mesh,
        compiler_params=pltpu.CompilerParams(needs_layout_passes=False),
        scratch_types=[
            pltpu.VMEM((EPP,), jnp.int32),
            pltpu.VMEM((EPP,), jnp.int32),
            pltpu.VMEM((RING,), jnp.int32),
            pltpu.VMEM((RING,), jnp.int32),
            pltpu.VMEM((RING,), jnp.int32),
            pltpu.VMEM((RING,), jnp.int32),
            pltpu.VMEM((NP,), jnp.float32),
            pltpu.VMEM((16,), jnp.int32),
        ],
    )
    def binner(srcf_hbm, dstf_hbm, srcb_hbm, dstb_hbm, cnts_hbm, deg_hbm,
               sv, dv, r0s, r0d, r1s, r1d, degl, cbuf):
        cid = lax.axis_index("c")
        sid = lax.axis_index("s")
        p = cid * NS + sid

        zero16 = jnp.zeros((16,), jnp.float32)
        iota16 = lax.iota(jnp.int32, 16)

        pltpu.sync_copy(srcf_hbm.at[p], sv)
        pltpu.sync_copy(dstf_hbm.at[p], dv)

        def zdeg(i, carry):
            degl[pl.ds(i * 16, 16)] = zero16
            return carry

        lax.fori_loop(0, NP // 16, zdeg, 0)

        rings = ((r0s, r0d), (r1s, r1d))

        def flush_full(b, nf):
            rs, rd = rings[b]
            o = pl.multiple_of(nf * FLUSH, FLUSH)
            pltpu.sync_copy(rs.at[pl.ds(0, FLUSH)],
                            srcb_hbm.at[b, p, pl.ds(o, FLUSH)])
            pltpu.sync_copy(rd.at[pl.ds(0, FLUSH)],
                            dstb_hbm.at[b, p, pl.ds(o, FLUSH)])
            rs[pl.ds(0, 16)] = rs[pl.ds(FLUSH, 16)]
            rd[pl.ds(0, 16)] = rd[pl.ds(FLUSH, 16)]

        def vec_body(i, carry):
            off0, nf0, off1, nf1 = carry
            s = sv[pl.ds(i * 16, 16)]
            d = dv[pl.ds(i * 16, 16)]

            cnt, last = plsc.scan_count(d)
            plsc.addupdate_scatter(degl, [d], cnt.astype(jnp.float32),
                                   mask=last)

            m0 = d < HALF
            m1 = jnp.logical_not(m0)
            c0 = plsc.cumsum(m0.astype(jnp.int32))
            c1 = plsc.cumsum(m1.astype(jnp.int32))
            n0 = jnp.max(c0)
            n1 = jnp.max(c1)
            pos0 = c0 + (off0 - 1)
            pos1 = c1 + (off1 - 1)
            plsc.store_scatter(r0s, [pos0], s, mask=m0)
            plsc.store_scatter(r0d, [pos0], d, mask=m0)
            plsc.store_scatter(r1s, [pos1], s, mask=m1)
            plsc.store_scatter(r1d, [pos1], d - HALF, mask=m1)
            off0 = off0 + n0
            off1 = off1 + n1

            fl0 = off0 >= FLUSH

            @pl.when(fl0)
            def _():
                flush_full(0, nf0)

            nf0 = nf0 + fl0.astype(jnp.int32)
            off0 = off0 - FLUSH * fl0.astype(jnp.int32)

            fl1 = off1 >= FLUSH

            @pl.when(fl1)
            def _():
                flush_full(1, nf1)

            nf1 = nf1 + fl1.astype(jnp.int32)
            off1 = off1 - FLUSH * fl1.astype(jnp.int32)

            return (off0, nf0, off1, nf1)

        z = jnp.int32(0)
        off0, nf0, off1, nf1 = lax.fori_loop(
            0, EPP // 16, vec_body, (z, z, z, z))

        def finalize(b, off, nf):
            rs, rd = rings[b]
            npad = ((SEG - off % SEG) % SEG + 15) // 16

            def pad_body(k, carry):
                ppos = iota16 + (off + k * 16)
                plsc.store_scatter(rs, [ppos], jnp.zeros((16,), jnp.int32))
                plsc.store_scatter(rd, [ppos],
                                   jnp.full((16,), TRASH, jnp.int32))
                return carry

            lax.fori_loop(0, npad, pad_body, 0)
            nrem = (off + SEG - 1) // SEG
            for k in range(FLUSH // SEG):
                @pl.when(k < nrem)
                def _():
                    o = pl.multiple_of(nf * FLUSH, SEG) + k * SEG
                    pltpu.sync_copy(rs.at[pl.ds(k * SEG, SEG)],
                                    srcb_hbm.at[b, p, pl.ds(o, SEG)])
                    pltpu.sync_copy(rd.at[pl.ds(k * SEG, SEG)],
                                    dstb_hbm.at[b, p, pl.ds(o, SEG)])

            return nf * (FLUSH // SEG) + nrem

        nseg0 = finalize(0, off0, nf0)
        nseg1 = finalize(1, off1, nf1)

        cv = (jnp.where(iota16 == 0, nseg0, 0)
              + jnp.where(iota16 == 1, nseg1, 0))
        cbuf[pl.ds(0, 16)] = cv
        pltpu.sync_copy(cbuf, cnts_hbm.at[p])
        pltpu.sync_copy(degl, deg_hbm.at[p])

    return binner(srcf, dstf)


def _sc_scatter(xg, srcb5, dstb5, cnts):
    mesh = plsc.VectorSubcoreMesh(core_axis_name="c", subcore_axis_name="s")

    @functools.partial(
        pl.kernel,
        out_type=jax.ShapeDtypeStruct((NC * NPC, D), jnp.float32),
        mesh=mesh,
        compiler_params=pltpu.CompilerParams(needs_layout_passes=False),
        scratch_types=[
            pltpu.VMEM((4 * SEG // KC, KC), jnp.int32),
            pltpu.VMEM((4 * SEG // KC, KC), jnp.int32),
            pltpu.VMEM((4 * SEG // KC, KC), jnp.int32),
            pltpu.VMEM((4 * SEG // KC, KC), jnp.int32),
            pltpu.VMEM((KC, D), jnp.float32),
            pltpu.VMEM((KC, D), jnp.float32),
            pltpu.VMEM((16,), jnp.int32),
            pltpu.VMEM_SHARED((NPC, D), jnp.float32),
            pltpu.SemaphoreType.DMA,
            pltpu.SemaphoreType.DMA,
            pltpu.SemaphoreType.DMA,
        ],
    )
    def scat(xg_hbm, src_hbm, dst_hbm, cnts_hbm, agg_hbm,
             ssA, ddA, ssB, ddB, rows0, rows1, cbuf, acc, sem0, sem1, semi):
        cid = lax.axis_index("c")
        sid = lax.axis_index("s")
        rows = (rows0, rows1)
        sems = (sem0, sem1)
        slots = ((ssA, ddA), (ssB, ddB))

        zero16 = jnp.zeros((16,), jnp.float32)
        iota16 = lax.iota(jnp.int32, 16)
        rpt = NPC // NS

        def zrow(i, carry):
            for c0 in range(0, D, 16):
                rows0[i, pl.ds(c0, 16)] = zero16
            return carry

        lax.fori_loop(0, KC, zrow, 0)
        for k in range(rpt // KC):
            pltpu.sync_copy(rows0, acc.at[pl.ds(sid * rpt + k * KC, KC)])
        pltpu.sync_copy(rows0.at[pl.ds(0, rpt % KC)],
                        acc.at[pl.ds(sid * rpt + rpt - rpt % KC, rpt % KC)])
        plsc.subcore_barrier()

        def g_start(idx, buf, sem):
            pltpu.make_async_copy(xg_hbm.at[idx], buf, sem).start()

        def g_wait(buf, sem):
            pltpu.make_async_copy(xg_hbm.at[ssA.at[0]], buf, sem).wait()

        NK = SEG // KC
        BC = 4 * NK

        for pp in range(2):
            p = 2 * sid + pp
            pltpu.sync_copy(cnts_hbm.at[p], cbuf)
            cv = cbuf[pl.ds(0, 16)]
            nseg = jnp.max(jnp.where(iota16 == cid, cv, 0))

            def idx_start(blk, slot):
                pltpu.make_async_copy(src_hbm.at[cid, p, blk],
                                      slots[slot][0], semi).start()
                pltpu.make_async_copy(dst_hbm.at[cid, p, blk],
                                      slots[slot][1], semi).start()

            def idx_wait():
                pltpu.make_async_copy(src_hbm.at[cid, p, 0], ssA, semi).wait()
                pltpu.make_async_copy(dst_hbm.at[cid, p, 0], ddA, semi).wait()

            @pl.when(nseg > 0)
            def _():
                idx_start(0, 0)
                idx_wait()
                g_start(ssA.at[0], rows0, sem0)

            def pair_body(j, carry):
                for half in range(2):
                    blk = 2 * j + half
                    s_s, d_s = slots[half]
                    nslot = 1 - half
                    nxt_blk = 4 * blk + 4 < nseg

                    @pl.when(4 * blk < nseg)
                    def _():
                        for sg_loc in range(4):
                            sg = 4 * blk + sg_loc

                            @pl.when(sg < nseg)
                            def _():
                                if sg_loc == 0:
                                    @pl.when(nxt_blk)
                                    def _():
                                        idx_start(blk + 1, nslot)
                                for k in range(NK):
                                    ci = 4 * sg_loc + k
                                    if ci < BC - 1:
                                        if k < NK - 1:
                                            g_start(s_s.at[ci + 1],
                                                    rows[(ci + 1) % 2],
                                                    sems[(ci + 1) % 2])
                                        else:
                                            @pl.when(sg + 1 < nseg)
                                            def _():
                                                g_start(s_s.at[ci + 1],
                                                        rows[(ci + 1) % 2],
                                                        sems[(ci + 1) % 2])
                                    else:
                                        @pl.when(nxt_blk)
                                        def _():
                                            idx_wait()
                                            g_start(slots[nslot][0].at[0],
                                                    rows[(ci + 1) % 2],
                                                    sems[(ci + 1) % 2])
                                    g_wait(rows[ci % 2], sems[ci % 2])
                                    pltpu.sync_copy(rows[ci % 2],
                                                    acc.at[d_s.at[ci]],
                                                    add=True)
                return carry

            nblk = (nseg + 3) // 4
            lax.fori_loop(0, (nblk + 1) // 2, pair_body, 0)

        plsc.subcore_barrier()
        pltpu.sync_copy(acc.at[pl.ds(sid * rpt, rpt)],
                        agg_hbm.at[pl.ds(cid * NPC + sid * rpt, rpt)])

    return scat(xg, srcb5, dstb5, cnts)


def _dense_layer(agg, deg, xg, WlT, WcT, b, relu):
    R = 256
    BPC = HALF // R

    def body(agg_ref, deg_ref, x_ref, wl_ref, wc_ref, b_ref, o_ref):
        a = agg_ref[...]
        ones = jnp.ones((NW, 1), jnp.float32)
        deg_col = lax.dot_general(
            deg_ref[...], ones, (((0,), (0,)), ((), ())),
            preferred_element_type=jnp.float32)
        mean = a / jnp.maximum(deg_col, 1.0)
        h = jnp.dot(mean, wl_ref[...], preferred_element_type=jnp.float32)
        h = h + jnp.dot(x_ref[...], wc_ref[...],
                        preferred_element_type=jnp.float32)
        h = h + b_ref[...]
        if relu:
            h = jnp.maximum(h, 0.0)
        o_ref[...] = h

    return pl.pallas_call(
        body,
        grid=(NP // R,),
        in_specs=[
            pl.BlockSpec((R, D),
                         lambda i: ((i // BPC) * (NPC // R) + i % BPC, 0)),
            pl.BlockSpec((NW, R), lambda i: (0, i)),
            pl.BlockSpec((R, D), lambda i: (i, 0)),
            pl.BlockSpec((D, D), lambda i: (0, 0)),
            pl.BlockSpec((D, D), lambda i: (0, 0)),
            pl.BlockSpec((1, D), lambda i: (0, 0)),
        ],
        out_specs=pl.BlockSpec((R, D), lambda i: (i, 0)),
        out_shape=jax.ShapeDtypeStruct((NP, D), jnp.float32),
    )(agg, deg, xg, WlT, WcT, b)


def kernel(x, edge_index, edge_attr, edge_weight,
           Wl1, bl1, Wr1, L1W, L1b, Wl2, bl2, Wr2, L2W, L2b):
    srcf = edge_index[0].reshape(NW, EPP)
    dstf = edge_index[1].reshape(NW, EPP)

    xg = jnp.pad(x, ((0, NP - N), (0, 0)))

    Wl1T = Wl1.T
    Wc1T = (Wr1 + L1W).T
    b1 = (bl1 + L1b).reshape(1, D)
    Wl2T = Wl2.T
    Wc2T = (Wr2 + L2W).T
    b2 = (bl2 + L2b).reshape(1, D)

    srcb, dstb, cnts, deg = _bin_edges(srcf, dstf)
    srcb5 = srcb.reshape(NC, NW, SEGS // 4, 4 * SEG // KC, KC)
    dstb5 = dstb.reshape(NC, NW, SEGS // 4, 4 * SEG // KC, KC)

    agg1 = _sc_scatter(xg, srcb5, dstb5, cnts)
    hg = _dense_layer(agg1, deg, xg, Wl1T, Wc1T, b1, relu=True)
    agg2 = _sc_scatter(hg, srcb5, dstb5, cnts)
    out = _dense_layer(agg2, deg, hg, Wl2T, Wc2T, b2, relu=False)
    return out[:N]


# speedup vs baseline: 1.8460x; 1.8460x over previous
import functools

import jax
import jax.numpy as jnp
from jax import lax
from jax.experimental import pallas as pl
from jax.experimental.pallas import tpu as pltpu
from jax.experimental.pallas import tpu_sc as plsc

N = 10000
E = 320000
D = 128
NP = 10240
NS = 16
EPW = E // NS
K = 80
G = 25
NG = EPW // (K * G)
RPT = NP // NS


def _sc_scatter(xg, src4, dst4, with_deg):
    mesh = plsc.VectorSubcoreMesh(
        core_axis_name="c", subcore_axis_name="s", num_cores=1)
    out_type = [jax.ShapeDtypeStruct((NP, D), jnp.float32)]
    if with_deg:
        out_type.append(jax.ShapeDtypeStruct((NS, NP), jnp.float32))

    @functools.partial(
        pl.kernel,
        out_type=out_type,
        mesh=mesh,
        compiler_params=pltpu.CompilerParams(needs_layout_passes=False),
        scratch_types=[
            pltpu.VMEM((G, K), jnp.int32),
            pltpu.VMEM((G, K), jnp.int32),
            pltpu.VMEM((G, K), jnp.int32),
            pltpu.VMEM((G, K), jnp.int32),
            pltpu.VMEM((K, D), jnp.float32),
            pltpu.VMEM((K, D), jnp.float32),
            pltpu.VMEM((NP,), jnp.float32),
            pltpu.VMEM_SHARED((NP, D), jnp.float32),
            pltpu.SemaphoreType.DMA,
            pltpu.SemaphoreType.DMA,
            pltpu.SemaphoreType.DMA,
        ],
    )
    def scat(xg_hbm, src_hbm, dst_hbm, agg_hbm, *rest):
        if with_deg:
            deg_hbm = rest[0]
            rest = rest[1:]
        (srcA, dstA, srcB, dstB, rows0, rows1, degl, acc,
         sem0, sem1, semi) = rest
        rows = (rows0, rows1)
        sems = (sem0, sem1)
        slots = ((srcA, dstA), (srcB, dstB))
        sid = lax.axis_index("s")

        zero16 = jnp.zeros((16,), jnp.float32)

        def zrow(i, carry):
            for c0 in range(0, D, 16):
                rows0[i, pl.ds(c0, 16)] = zero16
            return carry

        lax.fori_loop(0, K, zrow, 0)
        for k in range(RPT // K):
            pltpu.sync_copy(rows0, acc.at[pl.ds(sid * RPT + k * K, K)])
        if with_deg:
            def zdeg(i, carry):
                degl[pl.ds(i * 16, 16)] = zero16
                return carry

            lax.fori_loop(0, NP // 16, zdeg, 0)
        plsc.subcore_barrier()

        def idx_start(g, slot):
            src_s, dst_s = slots[slot]
            pltpu.make_async_copy(src_hbm.at[sid, g], src_s, semi).start()
            pltpu.make_async_copy(dst_hbm.at[sid, g], dst_s, semi).start()

        def idx_wait():
            pltpu.make_async_copy(src_hbm.at[sid, 0], srcA, semi).wait()
            pltpu.make_async_copy(dst_hbm.at[sid, 0], dstA, semi).wait()

        def g_start(slot, ct, buf, sem):
            pltpu.make_async_copy(
                xg_hbm.at[slots[slot][0].at[ct]], buf, sem).start()

        def g_wait(buf, sem):
            pltpu.make_async_copy(xg_hbm.at[srcA.at[0]], buf, sem).wait()

        idx_start(0, 0)
        idx_wait()
        g_start(0, 0, rows0, sem0)

        def pair_body(j, carry):
            for half in range(2):
                g = 2 * j + half
                slot = half
                nslot = 1 - half
                have_next = g + 1 <= NG - 1
                par0 = half * (G % 2)
                for ct in range(G):
                    b = (par0 + ct) % 2
                    bn = (par0 + ct + 1) % 2
                    if ct == 0:
                        @pl.when(have_next)
                        def _():
                            idx_start(g + 1, nslot)
                    if ct == G - 2:
                        @pl.when(have_next)
                        def _():
                            idx_wait()
                    if ct < G - 1:
                        g_start(slot, ct + 1, rows[bn], sems[bn])
                    else:
                        @pl.when(have_next)
                        def _():
                            g_start(nslot, 0, rows[bn], sems[bn])
                    if with_deg:
                        dst_s = slots[slot][1]
                        for k in range(K // 16):
                            idx = dst_s[ct, pl.ds(k * 16, 16)]
                            cnt, last = plsc.scan_count(idx)
                            plsc.addupdate_scatter(
                                degl, [idx], cnt.astype(jnp.float32),
                                mask=last)
                    g_wait(rows[b], sems[b])
                    pltpu.sync_copy(
                        rows[b], acc.at[slots[slot][1].at[ct]], add=True)
            return carry

        lax.fori_loop(0, NG // 2, pair_body, 0)
        if with_deg:
            pltpu.sync_copy(degl, deg_hbm.at[sid])
        plsc.subcore_barrier()

        pltpu.sync_copy(acc.at[pl.ds(sid * RPT, RPT)],
                        agg_hbm.at[pl.ds(sid * RPT, RPT)])

    res = scat(xg, src4, dst4)
    if with_deg:
        return res[0], res[1]
    return res[0]


def _dense_layer(agg, deg, xg, WlT, WcT, b, relu):
    R = 256

    def body(agg_ref, deg_ref, x_ref, wl_ref, wc_ref, b_ref, o_ref):
        a = agg_ref[...]
        ones = jnp.ones((NS, 1), jnp.float32)
        deg_col = lax.dot_general(
            deg_ref[...], ones, (((0,), (0,)), ((), ())),
            preferred_element_type=jnp.float32)
        mean = a / jnp.maximum(deg_col, 1.0)
        h = jnp.dot(mean, wl_ref[...], preferred_element_type=jnp.float32)
        h = h + jnp.dot(x_ref[...], wc_ref[...],
                        preferred_element_type=jnp.float32)
        h = h + b_ref[...]
        if relu:
            h = jnp.maximum(h, 0.0)
        o_ref[...] = h

    return pl.pallas_call(
        body,
        grid=(NP // R,),
        in_specs=[
            pl.BlockSpec((R, D), lambda i: (i, 0)),
            pl.BlockSpec((NS, R), lambda i: (0, i)),
            pl.BlockSpec((R, D), lambda i: (i, 0)),
            pl.BlockSpec((D, D), lambda i: (0, 0)),
            pl.BlockSpec((D, D), lambda i: (0, 0)),
            pl.BlockSpec((1, D), lambda i: (0, 0)),
        ],
        out_specs=pl.BlockSpec((R, D), lambda i: (i, 0)),
        out_shape=jax.ShapeDtypeStruct((NP, D), jnp.float32),
    )(agg, deg, xg, WlT, WcT, b)


def kernel(x, edge_index, edge_attr, edge_weight,
           Wl1, bl1, Wr1, L1W, L1b, Wl2, bl2, Wr2, L2W, L2b):
    src4 = edge_index[0].reshape(NS, NG, G, K)
    dst4 = edge_index[1].reshape(NS, NG, G, K)

    xg = jnp.pad(x, ((0, NP - N), (0, 0)))

    Wl1T = Wl1.T
    Wc1T = (Wr1 + L1W).T
    b1 = (bl1 + L1b).reshape(1, D)
    Wl2T = Wl2.T
    Wc2T = (Wr2 + L2W).T
    b2 = (bl2 + L2b).reshape(1, D)

    agg1, deg = _sc_scatter(xg, src4, dst4, with_deg=True)
    hg = _dense_layer(agg1, deg, xg, Wl1T, Wc1T, b1, relu=True)
    agg2 = _sc_scatter(hg, src4, dst4, with_deg=False)
    out = _dense_layer(agg2, deg, hg, Wl2T, Wc2T, b2, relu=False)
    return out[:N]
